# baseline (speedup 1.0000x reference)
import functools

import jax
import jax.numpy as jnp
from jax import lax
from jax.experimental import pallas as pl
from jax.experimental.pallas import tpu as pltpu
from jax.experimental.pallas import tpu_sc as plsc

_NUM_SC_CORES = 2
_NUM_SC_SUBCORES = 16
_SEQ_PAD = 52


def _sc_gather_rows(table_f32, idx_flat):
    n_rows = idx_flat.shape[0]
    d = table_f32.shape[1]
    n_workers = _NUM_SC_CORES * _NUM_SC_SUBCORES
    rows_per_worker = n_rows // n_workers
    chunk = 1664
    assert rows_per_worker % (2 * chunk) == 0
    n_chunks = rows_per_worker // chunk
    out_rows = n_rows * d // 128
    mesh = plsc.VectorSubcoreMesh(
        core_axis_name="c",
        subcore_axis_name="s",
        num_cores=_NUM_SC_CORES,
        num_subcores=_NUM_SC_SUBCORES,
    )

    group = 128 // d
    qchunk = chunk // group

    @functools.partial(
        pl.kernel,
        mesh=mesh,
        out_type=jax.ShapeDtypeStruct((out_rows, 128), jnp.float32),
        scratch_types=[
            [pltpu.VMEM((chunk,), jnp.int32) for _ in range(2)],
            [
                [pltpu.VMEM((qchunk, d), jnp.float32) for _ in range(group)]
                for _ in range(2)
            ],
            pltpu.VMEM_SHARED((table_f32.shape[0], d), jnp.float32),
            [pltpu.SemaphoreType.DMA for _ in range(2)],
            [pltpu.SemaphoreType.DMA for _ in range(2)],
        ],
        compiler_params=pltpu.CompilerParams(use_tc_tiling_on_sc=False),
    )
    def gather_kernel(
        table_hbm, idx_hbm, out_hbm, idx_v, rows_vs, table_sh, gsem, wbsem
    ):
        wid = lax.axis_index("s") * _NUM_SC_CORES + lax.axis_index("c")
        base = wid * rows_per_worker

        @pl.when(lax.axis_index("s") == 0)
        def _():
            pltpu.sync_copy(table_hbm, table_sh)

        plsc.subcore_barrier()

        def gather_descs(c, b):
            return [
                pltpu.make_async_copy(
                    table_sh.at[idx_v[b].at[pl.ds(p * qchunk, qchunk)]],
                    rows_vs[b][p],
                    gsem[b],
                )
                for p in range(group)
            ]

        def wb_descs(c, b):
            row0 = (base + c * chunk) * d // 128
            return [
                pltpu.make_async_copy(
                    rows_vs[b][p],
                    out_hbm.at[pl.ds(row0, qchunk), pl.ds(p * d, d)],
                    wbsem[b],
                )
                for p in range(group)
            ]

        def load_and_gather(c, b):
            off = base + c * chunk
            pltpu.sync_copy(idx_hbm.at[pl.ds(off, chunk)], idx_v[b])
            for desc in gather_descs(c, b):
                desc.start()

        def finish_chunk(c, b):
            for desc in gather_descs(c, b):
                desc.wait()
            for desc in wb_descs(c, b):
                desc.start()

        for b in range(2):
            load_and_gather(b, b)
            finish_chunk(b, b)

        @pl.loop(2, n_chunks, step=2)
        def _(j):
            for b in range(2):
                c = j + b
                for desc in wb_descs(c - 2, b):
                    desc.wait()
                load_and_gather(c, b)
                finish_chunk(c, b)

        for b in range(2):
            for desc in wb_descs(n_chunks - 2 + b, b):
                desc.wait()

    n_windows = n_rows // chunk
    idx_re = jnp.zeros((n_rows,), jnp.int32)
    return gather_kernel(table_f32, idx_re)


def _tc_mlp(x_lin, w1_bf16, b1, w2, b2, n):
    k = w1_bf16.shape[0]
    hid = w1_bf16.shape[1]
    out = w2.shape[1]
    tile = 1024
    xrows = tile * k // 128

    def body(x_ref, w1_ref, b1_ref, w2_ref, b2_ref, o_ref):
        o_ref[...] = jnp.broadcast_to(x_ref[0, :out] + w2_ref[0, :out], (tile, out))
        return
        x = x_ref[...].reshape(tile, k).astype(jnp.bfloat16)
        h = jnp.dot(x, w1_ref[...], preferred_element_type=jnp.float32)
        h = jnp.maximum(h + b1_ref[...], 0.0)
        logits = jnp.dot(h, w2_ref[...], preferred_element_type=jnp.float32)
        logits = logits + b2_ref[...]
        m = jnp.max(logits, axis=-1, keepdims=True)
        e = jnp.exp(logits - m)
        o_ref[...] = e / jnp.sum(e, axis=-1, keepdims=True)

    return pl.pallas_call(
        body,
        grid=(n // tile,),
        in_specs=[
            pl.BlockSpec((xrows, 128), lambda i: (i, 0)),
            pl.BlockSpec((k, hid), lambda i: (0, 0)),
            pl.BlockSpec((1, hid), lambda i: (0, 0)),
            pl.BlockSpec((hid, out), lambda i: (0, 0)),
            pl.BlockSpec((1, out), lambda i: (0, 0)),
        ],
        out_specs=pl.BlockSpec((tile, out), lambda i: (i, 0)),
        out_shape=jax.ShapeDtypeStruct((n, out), jnp.float32),
        compiler_params=pltpu.CompilerParams(
            dimension_semantics=("arbitrary",),
        ),
    )(x_lin, w1_bf16, b1.reshape(1, hid), w2, b2.reshape(1, out))


def kernel(inputs, emb, W1, b1, W2, b2):
    b, seq = inputs.shape
    v, e = emb.shape
    hid = W1.shape[1]
    table = jnp.concatenate([emb, jnp.zeros((1, e), emb.dtype)], axis=0)
    idx_pad = jnp.concatenate(
        [inputs, jnp.full((b, _SEQ_PAD - seq), v, jnp.int32)], axis=1
    )
    idx_flat = idx_pad.reshape(-1)
    x_lin = _sc_gather_rows(table, idx_flat)
    w1p = jnp.concatenate(
        [W1, jnp.zeros(((_SEQ_PAD - seq) * e, hid), W1.dtype)], axis=0
    ).astype(jnp.bfloat16)
    return _tc_mlp(x_lin, w1p, b1, W2, b2, b)


# speedup vs baseline: 1.9235x; 1.9235x over previous
import functools

import jax
import jax.numpy as jnp
import numpy as np
from jax import lax
from jax.experimental import pallas as pl
from jax.experimental.pallas import tpu as pltpu
from jax.experimental.pallas import tpu_sc as plsc

_NUM_SC_CORES = 2
_NUM_SC_SUBCORES = 16
_SEQ_PAD = 52


def _sc_gather_rows(table_f32, idx128, seq_pad):
    b_total = idx128.shape[0]
    d = table_f32.shape[1]
    group = 128 // d
    n_workers = _NUM_SC_CORES * _NUM_SC_SUBCORES
    b_per_worker = b_total // n_workers
    bchunk = 32
    assert b_per_worker % (2 * bchunk) == 0
    n_chunks = b_per_worker // bchunk
    chunk = bchunk * seq_pad
    qchunk = chunk // group
    out_rows = b_total * seq_pad * d // 128
    mesh = plsc.VectorSubcoreMesh(
        core_axis_name="c",
        subcore_axis_name="s",
        num_cores=_NUM_SC_CORES,
        num_subcores=_NUM_SC_SUBCORES,
    )

    dmap_np = np.zeros((bchunk, 128), np.int32)
    for bb in range(bchunk):
        for j in range(64):
            if j < seq_pad:
                dmap_np[bb, j] = (qchunk // bchunk) * bchunk * (j % group) + (
                    seq_pad // group
                ) * bb + j // group
            else:
                dmap_np[bb, j] = chunk + (j - seq_pad) % 16
    dmap = jnp.asarray(dmap_np)

    @functools.partial(
        pl.kernel,
        mesh=mesh,
        out_type=jax.ShapeDtypeStruct((out_rows, 128), jnp.float32),
        scratch_types=[
            [pltpu.VMEM((bchunk, 128), jnp.int32) for _ in range(2)],
            [pltpu.VMEM((chunk + 16,), jnp.int32) for _ in range(2)],
            [
                [pltpu.VMEM((qchunk, d), jnp.float32) for _ in range(group)]
                for _ in range(2)
            ],
            pltpu.VMEM((bchunk, 128), jnp.int32),
            pltpu.VMEM_SHARED((table_f32.shape[0], d), jnp.float32),
            [pltpu.SemaphoreType.DMA for _ in range(2)],
            [pltpu.SemaphoreType.DMA for _ in range(2)],
        ],
        compiler_params=pltpu.CompilerParams(
            use_tc_tiling_on_sc=False, needs_layout_passes=False
        ),
    )
    def gather_kernel(
        table_hbm, idx_hbm, dmap_hbm, out_hbm,
        slab_v, idxre_v, rows_vs, dmap_v, table_sh, gsem, wbsem,
    ):
        wid = lax.axis_index("s") * _NUM_SC_CORES + lax.axis_index("c")
        b_base = wid * b_per_worker
        piece_base = b_base * seq_pad

        @pl.when(lax.axis_index("s") == 0)
        def _():
            pltpu.sync_copy(table_hbm, table_sh)

        pltpu.sync_copy(dmap_hbm, dmap_v)
        plsc.subcore_barrier()

        def gather_descs(c, b):
            return [
                pltpu.make_async_copy(
                    table_sh.at[idxre_v[b].at[pl.ds(p * qchunk, qchunk)]],
                    rows_vs[b][p],
                    gsem[b],
                )
                for p in range(group)
            ]

        def wb_descs(c, b):
            row0 = (piece_base + c * chunk) * d // 128
            return [
                pltpu.make_async_copy(
                    rows_vs[b][p],
                    out_hbm.at[pl.ds(row0, qchunk), pl.ds(p * d, d)],
                    wbsem[b],
                )
                for p in range(group)
            ]

        def load_and_gather(c, b):
            pltpu.sync_copy(
                idx_hbm.at[pl.ds(b_base + c * bchunk, bchunk)], slab_v[b]
            )

            @pl.loop(0, bchunk)
            def _(bb):
                for cc in range(4):
                    vals = slab_v[b][bb, pl.ds(cc * 16, 16)]
                    dsts = dmap_v[bb, pl.ds(cc * 16, 16)]
                    plsc.store_scatter(idxre_v[b], [dsts], vals)

            for desc in gather_descs(c, b):
                desc.start()

        def finish_chunk(c, b):
            for desc in gather_descs(c, b):
                desc.wait()
            for desc in wb_descs(c, b):
                desc.start()

        for b in range(2):
            load_and_gather(b, b)
            finish_chunk(b, b)

        @pl.loop(2, n_chunks, step=2)
        def _(j):
            for b in range(2):
                c = j + b
                for desc in wb_descs(c - 2, b):
                    desc.wait()
                load_and_gather(c, b)
                finish_chunk(c, b)

        for b in range(2):
            for desc in wb_descs(n_chunks - 2 + b, b):
                desc.wait()

    return gather_kernel(table_f32, idx128, dmap)


def _tc_mlp(x_lin, w1_bf16, b1, w2, b2, n):
    k = w1_bf16.shape[0]
    hid = w1_bf16.shape[1]
    out = w2.shape[1]
    tile = 1024
    xrows = tile * k // 128

    def body(x_ref, w1_ref, b1_ref, w2_ref, b2_ref, o_ref):
        x = x_ref[...].reshape(tile, k).astype(jnp.bfloat16)
        h = jnp.dot(x, w1_ref[...], preferred_element_type=jnp.float32)
        h = jnp.maximum(h + b1_ref[...], 0.0)
        logits = jnp.dot(h, w2_ref[...], preferred_element_type=jnp.float32)
        logits = logits + b2_ref[...]
        m = jnp.max(logits, axis=-1, keepdims=True)
        e = jnp.exp(logits - m)
        o_ref[...] = e / jnp.sum(e, axis=-1, keepdims=True)

    return pl.pallas_call(
        body,
        grid=(n // tile,),
        in_specs=[
            pl.BlockSpec((xrows, 128), lambda i: (i, 0)),
            pl.BlockSpec((k, hid), lambda i: (0, 0)),
            pl.BlockSpec((1, hid), lambda i: (0, 0)),
            pl.BlockSpec((hid, out), lambda i: (0, 0)),
            pl.BlockSpec((1, out), lambda i: (0, 0)),
        ],
        out_specs=pl.BlockSpec((tile, out), lambda i: (i, 0)),
        out_shape=jax.ShapeDtypeStruct((n, out), jnp.float32),
        compiler_params=pltpu.CompilerParams(
            dimension_semantics=("arbitrary",),
        ),
    )(x_lin, w1_bf16, b1.reshape(1, hid), w2, b2.reshape(1, out))


def kernel(inputs, emb, W1, b1, W2, b2):
    b, seq = inputs.shape
    v, e = emb.shape
    hid = W1.shape[1]
    table = jnp.concatenate([emb, jnp.zeros((16, e), emb.dtype)], axis=0)
    pad_vals = jnp.broadcast_to(
        v + (jnp.arange(128 - seq, dtype=jnp.int32) % 16), (b, 128 - seq)
    )
    idx128 = jnp.concatenate([inputs, pad_vals], axis=1)
    x_lin = _sc_gather_rows(table, idx128, _SEQ_PAD)
    w1p = jnp.concatenate(
        [W1, jnp.zeros(((_SEQ_PAD - seq) * e, hid), W1.dtype)], axis=0
    ).astype(jnp.bfloat16)
    return _tc_mlp(x_lin, w1p, b1, W2, b2, b)
